# Initial kernel scaffold; baseline (speedup 1.0000x reference)
import jax
import jax.numpy as jnp
from jax.experimental import pallas as pl


def kernel(x, edge_index, W1_root, W1_neigh, b1, W2_root, W2_neigh, b2):
    raise NotImplementedError("write your pallas kernel here")


# speedup vs baseline: 5.5172x; 5.5172x over previous
import functools

import jax
import jax.numpy as jnp
from jax import lax
from jax.experimental import pallas as pl
from jax.experimental.pallas import tpu as pltpu
from jax.experimental.pallas import tpu_sc as plsc

N = 10000
NP = 10240
E = 320000
D = 128
DH = D // 2

NC = 2
NS = 16
EPT = E // NS
K = 80
NCHUNK = EPT // K
RPT = NP // NS
RCH = 128
NRC = RPT // RCH


def _make_seg_sum(with_deg: bool):
    mesh = plsc.VectorSubcoreMesh(core_axis_name="c", subcore_axis_name="s")
    out_type = [jax.ShapeDtypeStruct((NC, NP, DH), jnp.float32)]
    scratch = [
        pltpu.VMEM((NCHUNK, K), jnp.int32),
        pltpu.VMEM((NCHUNK, K), jnp.int32),
        pltpu.VMEM((K, DH), jnp.float32),
        pltpu.VMEM((RCH, DH), jnp.float32),
        pltpu.VMEM_SHARED((NP, DH), jnp.float32),
        pltpu.SemaphoreType.DMA,
    ]
    if with_deg:
        out_type.append(jax.ShapeDtypeStruct((NP, 16), jnp.float32))
        scratch += [
            pltpu.VMEM((K, 16), jnp.float32),
            pltpu.VMEM((RCH, 16), jnp.float32),
            pltpu.VMEM_SHARED((NP, 16), jnp.float32),
        ]

    @functools.partial(
        pl.kernel, mesh=mesh, out_type=tuple(out_type),
        scratch_types=scratch,
        compiler_params=pltpu.CompilerParams(use_tc_tiling_on_sc=False),
    )
    def seg_sum(x_hbm, src_hbm, dst_hbm, z_hbm, zd_hbm, ones_hbm, *rest):
        if with_deg:
            (out_hbm, deg_hbm, src_v, dst_v, gbuf, cbuf, acc, sem,
             ones_v, cbuf16, accd) = rest
        else:
            out_hbm, src_v, dst_v, gbuf, cbuf, acc, sem = rest
        c = lax.axis_index("c")
        s = lax.axis_index("s")
        on_deg_core = c == 0

        pltpu.sync_copy(src_hbm.at[s], src_v)
        pltpu.sync_copy(dst_hbm.at[s], dst_v)
        pltpu.sync_copy(z_hbm, cbuf)
        if with_deg:
            pltpu.sync_copy(ones_hbm, ones_v)
            pltpu.sync_copy(zd_hbm, cbuf16)

        for t in range(NRC):
            row0 = s * RPT + t * RCH
            pltpu.sync_copy(cbuf, acc.at[pl.ds(row0, RCH)])
            if with_deg:
                @pl.when(on_deg_core)
                def _():
                    pltpu.sync_copy(cbuf16, accd.at[pl.ds(row0, RCH)])
        plsc.subcore_barrier()

        def body(ci, carry):
            pltpu.async_copy(x_hbm.at[c].at[src_v.at[ci]], gbuf, sem).wait()
            pltpu.sync_copy(gbuf, acc.at[dst_v.at[ci]], add=True)
            if with_deg:
                @pl.when(on_deg_core)
                def _():
                    pltpu.sync_copy(ones_v, accd.at[dst_v.at[ci]], add=True)
            return carry

        lax.fori_loop(0, NCHUNK, body, 0)
        plsc.subcore_barrier()

        for t in range(NRC):
            row0 = s * RPT + t * RCH
            pltpu.sync_copy(acc.at[pl.ds(row0, RCH)], cbuf)
            pltpu.sync_copy(cbuf, out_hbm.at[c, pl.ds(row0, RCH)])
            if with_deg:
                @pl.when(on_deg_core)
                def _():
                    pltpu.sync_copy(accd.at[pl.ds(row0, RCH)], cbuf16)
                    pltpu.sync_copy(cbuf16, deg_hbm.at[pl.ds(row0, RCH)])

    def run(xT, src, dst):
        z = jnp.zeros((RCH, DH), jnp.float32)
        zd = jnp.zeros((RCH, 16), jnp.float32)
        ones = jnp.ones((K, 16), jnp.float32)
        return seg_sum(xT, src, dst, z, zd, ones)

    return run


_seg_sum_deg = _make_seg_sum(True)
_seg_sum = _make_seg_sum(False)


def _tc_body(x_ref, p_ref, d_ref, wr_ref, wn_ref, b_ref, o_ref, *, relu):
    ssum = jnp.concatenate([p_ref[0, :N], p_ref[1, :N]], axis=-1)
    deg = d_ref[:N, :1]
    dinv = 1.0 / jnp.maximum(deg, 1.0)
    mean = ssum * dinv
    acc = (
        jnp.dot(x_ref[...], wr_ref[...], preferred_element_type=jnp.float32)
        + jnp.dot(mean, wn_ref[...], preferred_element_type=jnp.float32)
        + b_ref[...]
    )
    if relu:
        acc = jnp.maximum(acc, 0.0)
    o_ref[...] = acc


def _tc_layer(x, parts, deg, w_root, w_neigh, b, relu):
    return pl.pallas_call(
        functools.partial(_tc_body, relu=relu),
        out_shape=jax.ShapeDtypeStruct((N, D), jnp.float32),
    )(x, parts, deg, w_root, w_neigh, b.reshape(1, D))


def _col_split(x):
    return jnp.stack([x[:, :DH], x[:, DH:]])


def kernel(x, edge_index, W1_root, W1_neigh, b1, W2_root, W2_neigh, b2):
    src = edge_index[0].reshape(NS, NCHUNK, K)
    dst = edge_index[1].reshape(NS, NCHUNK, K)
    parts1, deg = _seg_sum_deg(_col_split(x), src, dst)
    h = _tc_layer(x, parts1, deg, W1_root, W1_neigh, b1, relu=True)
    (parts2,) = _seg_sum(_col_split(h), src, dst)
    return _tc_layer(h, parts2, deg, W2_root, W2_neigh, b2, relu=False)


# speedup vs baseline: 7.4591x; 1.3520x over previous
import functools

import jax
import jax.numpy as jnp
from jax import lax
from jax.experimental import pallas as pl
from jax.experimental.pallas import tpu as pltpu
from jax.experimental.pallas import tpu_sc as plsc

N = 10000
NP = 10240
E = 320000
D = 128
DH = D // 2

NC = 2
NS = 16
EPT = E // NS
K = 80
NCHUNK = EPT // K
RPT = NP // NS
RCH = 128
NRC = RPT // RCH
U = 2


def _make_seg_sum(with_deg: bool):
    mesh = plsc.VectorSubcoreMesh(core_axis_name="c", subcore_axis_name="s")
    out_type = [jax.ShapeDtypeStruct((NC, NP, DH), jnp.float32)]
    scratch = [
        pltpu.VMEM((NCHUNK, K), jnp.int32),
        pltpu.VMEM((NCHUNK, K), jnp.int32),
        pltpu.VMEM((RCH, DH), jnp.float32),
        pltpu.VMEM_SHARED((NP, DH), jnp.float32),
    ]
    scratch += [pltpu.VMEM((K, DH), jnp.float32) for _ in range(U)]
    scratch += [pltpu.SemaphoreType.DMA for _ in range(U)]
    if with_deg:
        out_type.append(jax.ShapeDtypeStruct((NC, NP, 16), jnp.float32))
        scratch += [
            pltpu.VMEM((K, 16), jnp.float32),
            pltpu.VMEM((RCH, 16), jnp.float32),
            pltpu.VMEM_SHARED((NP, 16), jnp.float32),
        ]

    @functools.partial(
        pl.kernel, mesh=mesh, out_type=tuple(out_type),
        scratch_types=scratch,
        compiler_params=pltpu.CompilerParams(use_tc_tiling_on_sc=False),
    )
    def seg_sum(x_hbm, src_hbm, dst_hbm, z_hbm, zd_hbm, ones_hbm, *rest):
        if with_deg:
            (out_hbm, deg_hbm, src_v, dst_v, cbuf, acc, *gb) = rest
            gbufs, sems = gb[:U], gb[U:2 * U]
            ones_v, cbuf16, accd = gb[2 * U:]
        else:
            out_hbm, src_v, dst_v, cbuf, acc, *gb = rest
            gbufs, sems = gb[:U], gb[U:]
        c = lax.axis_index("c")
        s = lax.axis_index("s")

        pltpu.sync_copy(src_hbm.at[s], src_v)
        pltpu.sync_copy(dst_hbm.at[s], dst_v)
        pltpu.sync_copy(z_hbm, cbuf)
        if with_deg:
            pltpu.sync_copy(ones_hbm, ones_v)
            pltpu.sync_copy(zd_hbm, cbuf16)

        for t in range(NRC):
            row0 = s * RPT + t * RCH
            pltpu.sync_copy(cbuf, acc.at[pl.ds(row0, RCH)])
            if with_deg:
                pltpu.sync_copy(cbuf16, accd.at[pl.ds(row0, RCH)])
        plsc.subcore_barrier()

        def body(j, carry):
            base = j * U
            handles = [
                pltpu.async_copy(
                    x_hbm.at[c].at[src_v.at[base + u]], gbufs[u], sems[u])
                for u in range(U)
            ]
            for u in range(U):
                handles[u].wait()
                pltpu.sync_copy(gbufs[u], acc.at[dst_v.at[base + u]],
                                add=True)
                if with_deg:
                    @pl.when(((base + u) % NC) == c)
                    def _():
                        pltpu.sync_copy(ones_v, accd.at[dst_v.at[base + u]],
                                        add=True)
            return carry

        lax.fori_loop(0, NCHUNK // U, body, 0)
        plsc.subcore_barrier()

        for t in range(NRC):
            row0 = s * RPT + t * RCH
            pltpu.sync_copy(acc.at[pl.ds(row0, RCH)], cbuf)
            pltpu.sync_copy(cbuf, out_hbm.at[c, pl.ds(row0, RCH)])
            if with_deg:
                pltpu.sync_copy(accd.at[pl.ds(row0, RCH)], cbuf16)
                pltpu.sync_copy(cbuf16, deg_hbm.at[c, pl.ds(row0, RCH)])

    def run(xT, src, dst):
        z = jnp.zeros((RCH, DH), jnp.float32)
        zd = jnp.zeros((RCH, 16), jnp.float32)
        ones = jnp.ones((K, 16), jnp.float32)
        return seg_sum(xT, src, dst, z, zd, ones)

    return run


_seg_sum_deg = _make_seg_sum(True)
_seg_sum = _make_seg_sum(False)


def _tc_body(x_ref, p_ref, d_ref, wr_ref, wn_ref, b_ref, o_ref, *, relu):
    ssum = jnp.concatenate([p_ref[0, :N], p_ref[1, :N]], axis=-1)
    deg = d_ref[0, :N, :1] + d_ref[1, :N, :1]
    dinv = 1.0 / jnp.maximum(deg, 1.0)
    mean = ssum * dinv
    acc = (
        jnp.dot(x_ref[...], wr_ref[...], preferred_element_type=jnp.float32)
        + jnp.dot(mean, wn_ref[...], preferred_element_type=jnp.float32)
        + b_ref[...]
    )
    if relu:
        acc = jnp.maximum(acc, 0.0)
    o_ref[...] = acc


def _tc_layer(x, parts, deg, w_root, w_neigh, b, relu):
    return pl.pallas_call(
        functools.partial(_tc_body, relu=relu),
        out_shape=jax.ShapeDtypeStruct((N, D), jnp.float32),
    )(x, parts, deg, w_root, w_neigh, b.reshape(1, D))


def _col_split(x):
    return jnp.stack([x[:, :DH], x[:, DH:]])


def kernel(x, edge_index, W1_root, W1_neigh, b1, W2_root, W2_neigh, b2):
    src = edge_index[0].reshape(NS, NCHUNK, K)
    dst = edge_index[1].reshape(NS, NCHUNK, K)
    parts1, deg = _seg_sum_deg(_col_split(x), src, dst)
    h = _tc_layer(x, parts1, deg, W1_root, W1_neigh, b1, relu=True)
    (parts2,) = _seg_sum(_col_split(h), src, dst)
    return _tc_layer(h, parts2, deg, W2_root, W2_neigh, b2, relu=False)


# speedup vs baseline: 8.7510x; 1.1732x over previous
import functools

import jax
import jax.numpy as jnp
from jax import lax
from jax.experimental import pallas as pl
from jax.experimental.pallas import tpu as pltpu
from jax.experimental.pallas import tpu_sc as plsc

N = 10000
NP = 10240
E = 320000
D = 128
DH = D // 2

NC = 2
NS = 16
EPT = E // NS
K = 80
NCHUNK = EPT // K
RPT = NP // NS
RCH = 128
NRC = RPT // RCH
U = 5


def _make_seg_sum(with_deg: bool):
    mesh = plsc.VectorSubcoreMesh(core_axis_name="c", subcore_axis_name="s")
    out_type = [jax.ShapeDtypeStruct((NC, NP, DH), jnp.float32)]
    scratch = [
        pltpu.VMEM((NCHUNK, K), jnp.int32),
        pltpu.VMEM((NCHUNK, K), jnp.int32),
        pltpu.VMEM((RCH, DH), jnp.float32),
        pltpu.VMEM_SHARED((NP, DH), jnp.float32),
    ]
    scratch += [pltpu.VMEM((K, DH), jnp.float32) for _ in range(U)]
    scratch += [pltpu.SemaphoreType.DMA for _ in range(U)]
    if with_deg:
        out_type.append(jax.ShapeDtypeStruct((NC, NP, 16), jnp.float32))
        scratch += [
            pltpu.VMEM((K, 16), jnp.float32),
            pltpu.VMEM((RCH, 16), jnp.float32),
            pltpu.VMEM_SHARED((NP, 16), jnp.float32),
        ]

    @functools.partial(
        pl.kernel, mesh=mesh, out_type=tuple(out_type),
        scratch_types=scratch,
        compiler_params=pltpu.CompilerParams(use_tc_tiling_on_sc=False),
    )
    def seg_sum(x_hbm, src_hbm, dst_hbm, z_hbm, zd_hbm, ones_hbm, *rest):
        if with_deg:
            (out_hbm, deg_hbm, src_v, dst_v, cbuf, acc, *gb) = rest
            gbufs, sems = gb[:U], gb[U:2 * U]
            ones_v, cbuf16, accd = gb[2 * U:]
        else:
            out_hbm, src_v, dst_v, cbuf, acc, *gb = rest
            gbufs, sems = gb[:U], gb[U:]
        c = lax.axis_index("c")
        s = lax.axis_index("s")

        pltpu.sync_copy(src_hbm.at[s], src_v)
        pltpu.sync_copy(dst_hbm.at[s], dst_v)
        pltpu.sync_copy(z_hbm, cbuf)
        if with_deg:
            pltpu.sync_copy(ones_hbm, ones_v)
            pltpu.sync_copy(zd_hbm, cbuf16)

        for t in range(NRC):
            row0 = s * RPT + t * RCH
            pltpu.sync_copy(cbuf, acc.at[pl.ds(row0, RCH)])
            if with_deg:
                pltpu.sync_copy(cbuf16, accd.at[pl.ds(row0, RCH)])
        plsc.subcore_barrier()

        def body(j, carry):
            base = j * U
            handles = [
                pltpu.async_copy(
                    x_hbm.at[c].at[src_v.at[base + u]], gbufs[u], sems[u])
                for u in range(U)
            ]
            for u in range(U):
                handles[u].wait()
                pltpu.sync_copy(gbufs[u], acc.at[dst_v.at[base + u]],
                                add=True)
                if with_deg:
                    @pl.when(((base + u) % NC) == c)
                    def _():
                        pltpu.sync_copy(ones_v, accd.at[dst_v.at[base + u]],
                                        add=True)
            return carry

        lax.fori_loop(0, NCHUNK // U, body, 0)
        plsc.subcore_barrier()

        for t in range(NRC):
            row0 = s * RPT + t * RCH
            pltpu.sync_copy(acc.at[pl.ds(row0, RCH)], cbuf)
            pltpu.sync_copy(cbuf, out_hbm.at[c, pl.ds(row0, RCH)])
            if with_deg:
                pltpu.sync_copy(accd.at[pl.ds(row0, RCH)], cbuf16)
                pltpu.sync_copy(cbuf16, deg_hbm.at[c, pl.ds(row0, RCH)])

    def run(xT, src, dst):
        z = jnp.zeros((RCH, DH), jnp.float32)
        zd = jnp.zeros((RCH, 16), jnp.float32)
        ones = jnp.ones((K, 16), jnp.float32)
        return seg_sum(xT, src, dst, z, zd, ones)

    return run


_seg_sum_deg = _make_seg_sum(True)
_seg_sum = _make_seg_sum(False)


def _tc_body(x_ref, p_ref, d_ref, wr_ref, wn_ref, b_ref, o_ref, *, relu):
    ssum = jnp.concatenate([p_ref[0, :N], p_ref[1, :N]], axis=-1)
    deg = d_ref[0, :N, :1] + d_ref[1, :N, :1]
    dinv = 1.0 / jnp.maximum(deg, 1.0)
    mean = ssum * dinv
    acc = (
        jnp.dot(x_ref[...], wr_ref[...], preferred_element_type=jnp.float32)
        + jnp.dot(mean, wn_ref[...], preferred_element_type=jnp.float32)
        + b_ref[...]
    )
    if relu:
        acc = jnp.maximum(acc, 0.0)
    o_ref[...] = acc


def _tc_layer(x, parts, deg, w_root, w_neigh, b, relu):
    return pl.pallas_call(
        functools.partial(_tc_body, relu=relu),
        out_shape=jax.ShapeDtypeStruct((N, D), jnp.float32),
    )(x, parts, deg, w_root, w_neigh, b.reshape(1, D))


def _col_split(x):
    return jnp.stack([x[:, :DH], x[:, DH:]])


def kernel(x, edge_index, W1_root, W1_neigh, b1, W2_root, W2_neigh, b2):
    src = edge_index[0].reshape(NS, NCHUNK, K)
    dst = edge_index[1].reshape(NS, NCHUNK, K)
    parts1, deg = _seg_sum_deg(_col_split(x), src, dst)
    h = _tc_layer(x, parts1, deg, W1_root, W1_neigh, b1, relu=True)
    (parts2,) = _seg_sum(_col_split(h), src, dst)
    return _tc_layer(h, parts2, deg, W2_root, W2_neigh, b2, relu=False)


# speedup vs baseline: 12.5174x; 1.4304x over previous
import functools

import jax
import jax.numpy as jnp
from jax import lax
from jax.experimental import pallas as pl
from jax.experimental.pallas import tpu as pltpu
from jax.experimental.pallas import tpu_sc as plsc

N = 10000
NP = 10240
E = 320000
D = 128
DH = D // 2

NC = 2
NS = 16
EPT = E // NS
K = 80
NCHUNK = EPT // K
RPT = NP // NS
RCH = 128
NRC = RPT // RCH
U = 5


def _make_seg_sum(with_deg: bool):
    mesh = plsc.VectorSubcoreMesh(core_axis_name="c", subcore_axis_name="s")
    out_type = [jax.ShapeDtypeStruct((NC, NP, DH), jnp.float32)]
    scratch = [
        pltpu.VMEM((NCHUNK, K), jnp.int32),
        pltpu.VMEM((NCHUNK, K), jnp.int32),
        pltpu.VMEM((RCH, DH), jnp.float32),
        pltpu.VMEM_SHARED((NP, DH), jnp.float32),
    ]
    scratch += [pltpu.VMEM((K, DH), jnp.float32) for _ in range(U)]
    scratch += [pltpu.SemaphoreType.DMA for _ in range(U)]
    if with_deg:
        out_type.append(jax.ShapeDtypeStruct((NC, NP, 16), jnp.float32))
        scratch += [
            pltpu.VMEM((K, 16), jnp.float32),
            pltpu.VMEM((RCH, 16), jnp.float32),
            pltpu.VMEM_SHARED((NP, 16), jnp.float32),
        ]

    @functools.partial(
        pl.kernel, mesh=mesh, out_type=tuple(out_type),
        scratch_types=scratch,
        compiler_params=pltpu.CompilerParams(use_tc_tiling_on_sc=False),
    )
    def seg_sum(x_hbm, src_hbm, dst_hbm, z_hbm, zd_hbm, ones_hbm, *rest):
        if with_deg:
            (out_hbm, deg_hbm, src_v, dst_v, cbuf, acc, *gb) = rest
            gbufs, sems = gb[:U], gb[U:2 * U]
            ones_v, cbuf16, accd = gb[2 * U:]
        else:
            out_hbm, src_v, dst_v, cbuf, acc, *gb = rest
            gbufs, sems = gb[:U], gb[U:]
        c = lax.axis_index("c")
        s = lax.axis_index("s")

        pltpu.sync_copy(src_hbm.at[s], src_v)
        pltpu.sync_copy(dst_hbm.at[s], dst_v)
        pltpu.sync_copy(z_hbm, cbuf)
        if with_deg:
            pltpu.sync_copy(ones_hbm, ones_v)
            pltpu.sync_copy(zd_hbm, cbuf16)

        for t in range(NRC):
            row0 = s * RPT + t * RCH
            pltpu.sync_copy(cbuf, acc.at[pl.ds(row0, RCH)])
            if with_deg:
                pltpu.sync_copy(cbuf16, accd.at[pl.ds(row0, RCH)])
        plsc.subcore_barrier()

        def gather(ci, u):
            return pltpu.async_copy(
                x_hbm.at[c].at[src_v.at[ci]], gbufs[u], sems[u])

        for u in range(U):
            gather(u, u)

        def body(j, carry):
            base = j * U
            for u in range(U):
                ci = base + u
                pltpu.make_async_copy(
                    x_hbm.at[c].at[src_v.at[ci]], gbufs[u], sems[u]).wait()
                pltpu.sync_copy(gbufs[u], acc.at[dst_v.at[ci]], add=True)
                if with_deg:
                    @pl.when((ci % NC) == c)
                    def _():
                        pltpu.sync_copy(ones_v, accd.at[dst_v.at[ci]],
                                        add=True)

                @pl.when(ci + U < NCHUNK)
                def _():
                    gather(ci + U, u)
            return carry

        lax.fori_loop(0, NCHUNK // U, body, 0)
        plsc.subcore_barrier()

        for t in range(NRC):
            row0 = s * RPT + t * RCH
            pltpu.sync_copy(acc.at[pl.ds(row0, RCH)], cbuf)
            pltpu.sync_copy(cbuf, out_hbm.at[c, pl.ds(row0, RCH)])
            if with_deg:
                pltpu.sync_copy(accd.at[pl.ds(row0, RCH)], cbuf16)
                pltpu.sync_copy(cbuf16, deg_hbm.at[c, pl.ds(row0, RCH)])

    def run(xT, src, dst):
        z = jnp.zeros((RCH, DH), jnp.float32)
        zd = jnp.zeros((RCH, 16), jnp.float32)
        ones = jnp.ones((K, 16), jnp.float32)
        return seg_sum(xT, src, dst, z, zd, ones)

    return run


_seg_sum_deg = _make_seg_sum(True)
_seg_sum = _make_seg_sum(False)


def _tc_body(x_ref, p_ref, d_ref, wr_ref, wn_ref, b_ref, o_ref, *, relu):
    ssum = jnp.concatenate([p_ref[0, :N], p_ref[1, :N]], axis=-1)
    deg = d_ref[0, :N, :1] + d_ref[1, :N, :1]
    dinv = 1.0 / jnp.maximum(deg, 1.0)
    mean = ssum * dinv
    acc = (
        jnp.dot(x_ref[...], wr_ref[...], preferred_element_type=jnp.float32)
        + jnp.dot(mean, wn_ref[...], preferred_element_type=jnp.float32)
        + b_ref[...]
    )
    if relu:
        acc = jnp.maximum(acc, 0.0)
    o_ref[...] = acc


def _tc_layer(x, parts, deg, w_root, w_neigh, b, relu):
    return pl.pallas_call(
        functools.partial(_tc_body, relu=relu),
        out_shape=jax.ShapeDtypeStruct((N, D), jnp.float32),
    )(x, parts, deg, w_root, w_neigh, b.reshape(1, D))


def _col_split(x):
    return jnp.stack([x[:, :DH], x[:, DH:]])


def kernel(x, edge_index, W1_root, W1_neigh, b1, W2_root, W2_neigh, b2):
    src = edge_index[0].reshape(NS, NCHUNK, K)
    dst = edge_index[1].reshape(NS, NCHUNK, K)
    parts1, deg = _seg_sum_deg(_col_split(x), src, dst)
    h = _tc_layer(x, parts1, deg, W1_root, W1_neigh, b1, relu=True)
    (parts2,) = _seg_sum(_col_split(h), src, dst)
    return _tc_layer(h, parts2, deg, W2_root, W2_neigh, b2, relu=False)


# speedup vs baseline: 14.4806x; 1.1568x over previous
import functools

import jax
import jax.numpy as jnp
from jax import lax
from jax.experimental import pallas as pl
from jax.experimental.pallas import tpu as pltpu
from jax.experimental.pallas import tpu_sc as plsc

N = 10000
NP = 10240
E = 320000
D = 128
DH = D // 2

NC = 2
NS = 16
EPT = E // NS
K = 80
NCHUNK = EPT // K
RPT = NP // NS
RCH = 128
NRC = RPT // RCH
U = 5


def _make_seg_sum(with_deg: bool):
    mesh = plsc.VectorSubcoreMesh(core_axis_name="c", subcore_axis_name="s")
    out_type = [jax.ShapeDtypeStruct((NP, D), jnp.float32)]
    scratch = [
        pltpu.VMEM((NCHUNK, K), jnp.int32),
        pltpu.VMEM((NCHUNK, K), jnp.int32),
        pltpu.VMEM((RCH, DH), jnp.float32),
        pltpu.VMEM_SHARED((NP, DH), jnp.float32),
    ]
    scratch += [pltpu.VMEM((K, DH), jnp.float32) for _ in range(U)]
    scratch += [pltpu.SemaphoreType.DMA for _ in range(U)]
    if with_deg:
        out_type.append(jax.ShapeDtypeStruct((NC, NP, 16), jnp.float32))
        scratch += [
            pltpu.VMEM((K, 16), jnp.float32),
            pltpu.VMEM((RCH, 16), jnp.float32),
            pltpu.VMEM_SHARED((NP, 16), jnp.float32),
        ]

    @functools.partial(
        pl.kernel, mesh=mesh, out_type=tuple(out_type),
        scratch_types=scratch,
        compiler_params=pltpu.CompilerParams(use_tc_tiling_on_sc=False),
    )
    def seg_sum(x_hbm, src_hbm, dst_hbm, z_hbm, zd_hbm, ones_hbm, *rest):
        if with_deg:
            (out_hbm, deg_hbm, src_v, dst_v, cbuf, acc, *gb) = rest
            gbufs, sems = gb[:U], gb[U:2 * U]
            ones_v, cbuf16, accd = gb[2 * U:]
        else:
            out_hbm, src_v, dst_v, cbuf, acc, *gb = rest
            gbufs, sems = gb[:U], gb[U:]
        c = lax.axis_index("c")
        s = lax.axis_index("s")

        pltpu.sync_copy(src_hbm.at[c].at[s], src_v)
        pltpu.sync_copy(dst_hbm.at[s], dst_v)
        pltpu.sync_copy(z_hbm, cbuf)
        if with_deg:
            pltpu.sync_copy(ones_hbm, ones_v)
            pltpu.sync_copy(zd_hbm, cbuf16)

        for t in range(NRC):
            row0 = s * RPT + t * RCH
            pltpu.sync_copy(cbuf, acc.at[pl.ds(row0, RCH)])
            if with_deg:
                pltpu.sync_copy(cbuf16, accd.at[pl.ds(row0, RCH)])
        plsc.subcore_barrier()

        def gather(ci, u):
            return pltpu.async_copy(
                x_hbm.at[src_v.at[ci]], gbufs[u], sems[u])

        for u in range(U):
            gather(u, u)

        def body(j, carry):
            base = j * U
            for u in range(U):
                ci = base + u
                pltpu.make_async_copy(
                    x_hbm.at[src_v.at[ci]], gbufs[u], sems[u]).wait()
                pltpu.sync_copy(gbufs[u], acc.at[dst_v.at[ci]], add=True)
                if with_deg:
                    @pl.when((ci % NC) == c)
                    def _():
                        pltpu.sync_copy(ones_v, accd.at[dst_v.at[ci]],
                                        add=True)

                @pl.when(ci + U < NCHUNK)
                def _():
                    gather(ci + U, u)
            return carry

        lax.fori_loop(0, NCHUNK // U, body, 0)
        plsc.subcore_barrier()

        for t in range(NRC):
            row0 = s * RPT + t * RCH
            pltpu.sync_copy(acc.at[pl.ds(row0, RCH)], cbuf)
            pltpu.sync_copy(
                cbuf, out_hbm.at[pl.ds(row0, RCH), pl.ds(c * DH, DH)])
            if with_deg:
                pltpu.sync_copy(accd.at[pl.ds(row0, RCH)], cbuf16)
                pltpu.sync_copy(cbuf16, deg_hbm.at[c, pl.ds(row0, RCH)])

    def run(x2, src2, dst):
        z = jnp.zeros((RCH, DH), jnp.float32)
        zd = jnp.zeros((RCH, 16), jnp.float32)
        ones = jnp.ones((K, 16), jnp.float32)
        return seg_sum(x2, src2, dst, z, zd, ones)

    return run


_seg_sum_deg = _make_seg_sum(True)
_seg_sum = _make_seg_sum(False)


def _tc_body(x_ref, p_ref, d_ref, wr_ref, wn_ref, b_ref, o_ref, *, relu):
    ssum = p_ref[:N]
    deg = d_ref[0, :N, :1] + d_ref[1, :N, :1]
    dinv = 1.0 / jnp.maximum(deg, 1.0)
    mean = ssum * dinv
    acc = (
        jnp.dot(x_ref[...], wr_ref[...], preferred_element_type=jnp.float32)
        + jnp.dot(mean, wn_ref[...], preferred_element_type=jnp.float32)
        + b_ref[...]
    )
    if relu:
        acc = jnp.maximum(acc, 0.0)
    o_ref[...] = acc


def _tc_layer(x, parts, deg, w_root, w_neigh, b, relu):
    return pl.pallas_call(
        functools.partial(_tc_body, relu=relu),
        out_shape=jax.ShapeDtypeStruct((N, D), jnp.float32),
    )(x, parts, deg, w_root, w_neigh, b.reshape(1, D))


def kernel(x, edge_index, W1_root, W1_neigh, b1, W2_root, W2_neigh, b2):
    s2 = edge_index[0] * 2
    src2 = jnp.stack([s2, s2 + 1]).reshape(NC, NS, NCHUNK, K)
    dst = edge_index[1].reshape(NS, NCHUNK, K)
    x2 = x.reshape(2 * N, DH)
    parts1, deg = _seg_sum_deg(x2, src2, dst)
    h = _tc_layer(x, parts1, deg, W1_root, W1_neigh, b1, relu=True)
    (parts2,) = _seg_sum(h.reshape(2 * N, DH), src2, dst)
    return _tc_layer(h, parts2, deg, W2_root, W2_neigh, b2, relu=False)


# speedup vs baseline: 15.7491x; 1.0876x over previous
import functools

import jax
import jax.numpy as jnp
from jax import lax
from jax.experimental import pallas as pl
from jax.experimental.pallas import tpu as pltpu
from jax.experimental.pallas import tpu_sc as plsc

N = 10000
NP = 10240
E = 320000
D = 128
DH = D // 2

NC = 2
NS = 16
EPT = E // NS
K = 80
NCHUNK = EPT // K
RPT = NP // NS
RCH = 128
NRC = RPT // RCH
U = 5


def _make_seg_sum(with_deg: bool):
    mesh = plsc.VectorSubcoreMesh(core_axis_name="c", subcore_axis_name="s")
    out_type = [jax.ShapeDtypeStruct((NP, D), jnp.float32)]
    scratch = [
        pltpu.VMEM((NCHUNK, K), jnp.int32),
        pltpu.VMEM((NCHUNK, K), jnp.int32),
        pltpu.VMEM((RCH, DH), jnp.float32),
        pltpu.VMEM_SHARED((NP, DH), jnp.float32),
    ]
    scratch += [pltpu.VMEM((K, DH), jnp.float32) for _ in range(U)]
    scratch += [pltpu.SemaphoreType.DMA for _ in range(U)]
    if with_deg:
        out_type.append(jax.ShapeDtypeStruct((NC, NP, 16), jnp.float32))
        scratch += [
            pltpu.VMEM((K, 16), jnp.float32),
            pltpu.VMEM((RCH, 16), jnp.float32),
            pltpu.VMEM_SHARED((NP, 16), jnp.float32),
        ]

    @functools.partial(
        pl.kernel, mesh=mesh, out_type=tuple(out_type),
        scratch_types=scratch,
        compiler_params=pltpu.CompilerParams(use_tc_tiling_on_sc=False),
    )
    def seg_sum(x_hbm, ei_hbm, z_hbm, zd_hbm, ones_hbm, *rest):
        if with_deg:
            (out_hbm, deg_hbm, src_v, dst_v, cbuf, acc, *gb) = rest
            gbufs, sems = gb[:U], gb[U:2 * U]
            ones_v, cbuf16, accd = gb[2 * U:]
        else:
            out_hbm, src_v, dst_v, cbuf, acc, *gb = rest
            gbufs, sems = gb[:U], gb[U:]
        c = lax.axis_index("c")
        s = lax.axis_index("s")

        pltpu.sync_copy(ei_hbm.at[0].at[s], src_v)
        pltpu.sync_copy(ei_hbm.at[1].at[s], dst_v)
        pltpu.sync_copy(z_hbm, cbuf)
        if with_deg:
            pltpu.sync_copy(ones_hbm, ones_v)
            pltpu.sync_copy(zd_hbm, cbuf16)

        for t in range(NRC):
            row0 = s * RPT + t * RCH
            pltpu.sync_copy(cbuf, acc.at[pl.ds(row0, RCH)])
            if with_deg:
                pltpu.sync_copy(cbuf16, accd.at[pl.ds(row0, RCH)])
        def fix_idx(r, carry):
            for l in range(K // 16):
                sl = pl.ds(l * 16, 16)
                src_v[r, sl] = src_v[r, sl] * 2 + c
            return carry

        lax.fori_loop(0, NCHUNK, fix_idx, 0)
        plsc.subcore_barrier()

        def gather(ci, u):
            return pltpu.async_copy(
                x_hbm.at[src_v.at[ci]], gbufs[u], sems[u])

        for u in range(U):
            gather(u, u)

        def body(j, carry):
            base = j * U
            for u in range(U):
                ci = base + u
                pltpu.make_async_copy(
                    x_hbm.at[src_v.at[ci]], gbufs[u], sems[u]).wait()
                pltpu.sync_copy(gbufs[u], acc.at[dst_v.at[ci]], add=True)
                if with_deg:
                    @pl.when((ci % NC) == c)
                    def _():
                        pltpu.sync_copy(ones_v, accd.at[dst_v.at[ci]],
                                        add=True)

                @pl.when(ci + U < NCHUNK)
                def _():
                    gather(ci + U, u)
            return carry

        lax.fori_loop(0, NCHUNK // U, body, 0)
        plsc.subcore_barrier()

        for t in range(NRC):
            row0 = s * RPT + t * RCH
            pltpu.sync_copy(acc.at[pl.ds(row0, RCH)], cbuf)
            pltpu.sync_copy(
                cbuf, out_hbm.at[pl.ds(row0, RCH), pl.ds(c * DH, DH)])
            if with_deg:
                pltpu.sync_copy(accd.at[pl.ds(row0, RCH)], cbuf16)
                pltpu.sync_copy(cbuf16, deg_hbm.at[c, pl.ds(row0, RCH)])

    def run(x2, ei):
        z = jnp.zeros((RCH, DH), jnp.float32)
        zd = jnp.zeros((RCH, 16), jnp.float32)
        ones = jnp.ones((K, 16), jnp.float32)
        return seg_sum(x2, ei, z, zd, ones)

    return run


_seg_sum_deg = _make_seg_sum(True)
_seg_sum = _make_seg_sum(False)


def _tc_body(x_ref, p_ref, d_ref, wr_ref, wn_ref, b_ref, o_ref, *, relu):
    ssum = p_ref[:N]
    deg = d_ref[0, :N, :1] + d_ref[1, :N, :1]
    dinv = 1.0 / jnp.maximum(deg, 1.0)
    mean = ssum * dinv
    acc = (
        jnp.dot(x_ref[...], wr_ref[...], preferred_element_type=jnp.float32)
        + jnp.dot(mean, wn_ref[...], preferred_element_type=jnp.float32)
        + b_ref[...]
    )
    if relu:
        acc = jnp.maximum(acc, 0.0)
    o_ref[...] = acc


def _tc_layer(x, parts, deg, w_root, w_neigh, b, relu):
    return pl.pallas_call(
        functools.partial(_tc_body, relu=relu),
        out_shape=jax.ShapeDtypeStruct((N, D), jnp.float32),
    )(x, parts, deg, w_root, w_neigh, b.reshape(1, D))


def kernel(x, edge_index, W1_root, W1_neigh, b1, W2_root, W2_neigh, b2):
    ei = edge_index.reshape(2, NS, NCHUNK, K)
    parts1, deg = _seg_sum_deg(x.reshape(2 * N, DH), ei)
    h = _tc_layer(x, parts1, deg, W1_root, W1_neigh, b1, relu=True)
    (parts2,) = _seg_sum(h.reshape(2 * N, DH), ei)
    return _tc_layer(h, parts2, deg, W2_root, W2_neigh, b2, relu=False)


# speedup vs baseline: 15.7683x; 1.0012x over previous
import functools

import jax
import jax.numpy as jnp
from jax import lax
from jax.experimental import pallas as pl
from jax.experimental.pallas import tpu as pltpu
from jax.experimental.pallas import tpu_sc as plsc

N = 10000
NP = 10240
E = 320000
D = 128
DH = D // 2

NC = 2
NS = 16
EPT = E // NS
K = 80
NCHUNK = EPT // K
RPT = NP // NS
RCH = 128
NRC = RPT // RCH


def _make_seg_sum(with_deg: bool):
    U = 5 if with_deg else 8
    mesh = plsc.VectorSubcoreMesh(core_axis_name="c", subcore_axis_name="s")
    out_type = [jax.ShapeDtypeStruct((NP, D), jnp.float32)]
    scratch = [
        pltpu.VMEM((NCHUNK, K), jnp.int32),
        pltpu.VMEM((NCHUNK, K), jnp.int32),
        pltpu.VMEM((RCH, DH), jnp.float32),
        pltpu.VMEM_SHARED((NP, DH), jnp.float32),
    ]
    scratch += [pltpu.VMEM((K, DH), jnp.float32) for _ in range(U)]
    scratch += [pltpu.SemaphoreType.DMA for _ in range(U)]
    if with_deg:
        out_type.append(jax.ShapeDtypeStruct((NC, NP, 16), jnp.float32))
        scratch += [
            pltpu.VMEM((K, 16), jnp.float32),
            pltpu.VMEM((RCH, 16), jnp.float32),
            pltpu.VMEM_SHARED((NP, 16), jnp.float32),
        ]

    @functools.partial(
        pl.kernel, mesh=mesh, out_type=tuple(out_type),
        scratch_types=scratch,
        compiler_params=pltpu.CompilerParams(use_tc_tiling_on_sc=False),
    )
    def seg_sum(x_hbm, ei_hbm, z_hbm, zd_hbm, ones_hbm, *rest):
        if with_deg:
            (out_hbm, deg_hbm, src_v, dst_v, cbuf, acc, *gb) = rest
            gbufs, sems = gb[:U], gb[U:2 * U]
            ones_v, cbuf16, accd = gb[2 * U:]
        else:
            out_hbm, src_v, dst_v, cbuf, acc, *gb = rest
            gbufs, sems = gb[:U], gb[U:]
        c = lax.axis_index("c")
        s = lax.axis_index("s")

        pltpu.sync_copy(ei_hbm.at[0].at[s], src_v)
        pltpu.sync_copy(ei_hbm.at[1].at[s], dst_v)
        pltpu.sync_copy(z_hbm, cbuf)
        if with_deg:
            pltpu.sync_copy(ones_hbm, ones_v)
            pltpu.sync_copy(zd_hbm, cbuf16)

        for t in range(NRC):
            row0 = s * RPT + t * RCH
            pltpu.sync_copy(cbuf, acc.at[pl.ds(row0, RCH)])
            if with_deg:
                pltpu.sync_copy(cbuf16, accd.at[pl.ds(row0, RCH)])
        def fix_idx(r, carry):
            for l in range(K // 16):
                sl = pl.ds(l * 16, 16)
                src_v[r, sl] = src_v[r, sl] * 2 + c
            return carry

        lax.fori_loop(0, NCHUNK, fix_idx, 0)
        plsc.subcore_barrier()

        def gather(ci, u):
            return pltpu.async_copy(
                x_hbm.at[src_v.at[ci]], gbufs[u], sems[u])

        for u in range(U):
            gather(u, u)

        def body(j, carry):
            base = j * U
            for u in range(U):
                ci = base + u
                pltpu.make_async_copy(
                    x_hbm.at[src_v.at[ci]], gbufs[u], sems[u]).wait()
                pltpu.sync_copy(gbufs[u], acc.at[dst_v.at[ci]], add=True)
                if with_deg:
                    @pl.when((ci % NC) == c)
                    def _():
                        pltpu.sync_copy(ones_v, accd.at[dst_v.at[ci]],
                                        add=True)

                @pl.when(ci + U < NCHUNK)
                def _():
                    gather(ci + U, u)
            return carry

        lax.fori_loop(0, NCHUNK // U, body, 0)
        for u in range(NCHUNK % U):
            ci = (NCHUNK // U) * U + u
            pltpu.make_async_copy(
                x_hbm.at[src_v.at[ci]], gbufs[u], sems[u]).wait()
            pltpu.sync_copy(gbufs[u], acc.at[dst_v.at[ci]], add=True)
            if with_deg:
                @pl.when((ci % NC) == c)
                def _():
                    pltpu.sync_copy(ones_v, accd.at[dst_v.at[ci]], add=True)
        plsc.subcore_barrier()

        for t in range(NRC):
            row0 = s * RPT + t * RCH
            pltpu.sync_copy(acc.at[pl.ds(row0, RCH)], cbuf)
            pltpu.sync_copy(
                cbuf, out_hbm.at[pl.ds(row0, RCH), pl.ds(c * DH, DH)])
            if with_deg:
                pltpu.sync_copy(accd.at[pl.ds(row0, RCH)], cbuf16)
                pltpu.sync_copy(cbuf16, deg_hbm.at[c, pl.ds(row0, RCH)])

    def run(x2, ei):
        z = jnp.zeros((RCH, DH), jnp.float32)
        zd = jnp.zeros((RCH, 16), jnp.float32)
        ones = jnp.ones((K, 16), jnp.float32)
        return seg_sum(x2, ei, z, zd, ones)

    return run


_seg_sum_deg = _make_seg_sum(True)
_seg_sum = _make_seg_sum(False)


def _tc_body(x_ref, p_ref, d_ref, wr_ref, wn_ref, b_ref, o_ref, *, relu):
    ssum = p_ref[:N]
    deg = d_ref[0, :N, :1] + d_ref[1, :N, :1]
    dinv = 1.0 / jnp.maximum(deg, 1.0)
    mean = ssum * dinv
    acc = (
        jnp.dot(x_ref[...], wr_ref[...], preferred_element_type=jnp.float32)
        + jnp.dot(mean, wn_ref[...], preferred_element_type=jnp.float32)
        + b_ref[...]
    )
    if relu:
        acc = jnp.maximum(acc, 0.0)
    o_ref[...] = acc


def _tc_layer(x, parts, deg, w_root, w_neigh, b, relu):
    return pl.pallas_call(
        functools.partial(_tc_body, relu=relu),
        out_shape=jax.ShapeDtypeStruct((N, D), jnp.float32),
    )(x, parts, deg, w_root, w_neigh, b.reshape(1, D))


def kernel(x, edge_index, W1_root, W1_neigh, b1, W2_root, W2_neigh, b2):
    ei = edge_index.reshape(2, NS, NCHUNK, K)
    parts1, deg = _seg_sum_deg(x.reshape(2 * N, DH), ei)
    h = _tc_layer(x, parts1, deg, W1_root, W1_neigh, b1, relu=True)
    (parts2,) = _seg_sum(h.reshape(2 * N, DH), ei)
    return _tc_layer(h, parts2, deg, W2_root, W2_neigh, b2, relu=False)
